# baseline (speedup 1.0000x reference)
import functools

import jax
import jax.numpy as jnp
from jax import lax
from jax.experimental import pallas as pl

B = 16
N = 20000
C = 80
K = 300
NR = 2000
NB = N // NR


def _reduce_body(l_ref, m_ref, c_ref):
    p = lax.logistic(l_ref[0])
    m = jnp.max(p, axis=1)
    ii = lax.broadcasted_iota(jnp.int32, (NR, C), 1)
    c1 = jnp.min(jnp.where(p == m[:, None], ii, C), axis=1)
    m_ref[0, 0, :] = m
    c_ref[0, 0, :] = c1


@jax.jit
def _dense_reduce(logits):
    grid = (B, NB)
    m, c1 = pl.pallas_call(
        _reduce_body,
        grid=grid,
        in_specs=[pl.BlockSpec((1, NR, C), lambda b, j: (b, j, 0))],
        out_specs=[
            pl.BlockSpec((1, 1, NR), lambda b, j: (b * NB + j, 0, 0)),
            pl.BlockSpec((1, 1, NR), lambda b, j: (b * NB + j, 0, 0)),
        ],
        out_shape=[
            jax.ShapeDtypeStruct((B * NB, 1, NR), jnp.float32),
            jax.ShapeDtypeStruct((B * NB, 1, NR), jnp.int32),
        ],
    )(logits)
    return m.reshape(B, N), c1.reshape(B, N)


def kernel(pred_logits, pred_boxes, orig_sizes):
    scores_all, labels = _dense_reduce(pred_logits)

    top_scores, top_idx = lax.top_k(scores_all, K)
    top_labels = jnp.take_along_axis(labels, top_idx, axis=1)
    top_boxes = jnp.take_along_axis(pred_boxes, top_idx[..., None], axis=1)
    cx, cy, w, h = (top_boxes[..., i] for i in range(4))
    box_xyxy = jnp.stack(
        [cx - 0.5 * w, cy - 0.5 * h, cx + 0.5 * w, cy + 0.5 * h], axis=-1)
    hw = orig_sizes.astype(jnp.float32)
    scale = jnp.stack([hw[:, 1], hw[:, 0], hw[:, 1], hw[:, 0]],
                      axis=-1)[:, None, :]
    return top_scores, top_labels, box_xyxy * scale


# speedup vs baseline: 1.0221x; 1.0221x over previous
import functools

import jax
import jax.numpy as jnp
from jax import lax
from jax.experimental import pallas as pl

B = 16
N = 20000
C = 80
K = 300
NR = 10000
NB = N // NR


def _reduce_body(l_ref, m_ref, c_ref):
    p = lax.logistic(l_ref[0])
    m = jnp.max(p, axis=1)
    ii = lax.broadcasted_iota(jnp.int32, (NR, C), 1)
    c1 = jnp.min(jnp.where(p == m[:, None], ii, C), axis=1)
    m_ref[0, 0, :] = m
    c_ref[0, 0, :] = c1


@jax.jit
def _dense_reduce(logits):
    grid = (B, NB)
    m, c1 = pl.pallas_call(
        _reduce_body,
        grid=grid,
        in_specs=[pl.BlockSpec((1, NR, C), lambda b, j: (b, j, 0))],
        out_specs=[
            pl.BlockSpec((1, 1, NR), lambda b, j: (b * NB + j, 0, 0)),
            pl.BlockSpec((1, 1, NR), lambda b, j: (b * NB + j, 0, 0)),
        ],
        out_shape=[
            jax.ShapeDtypeStruct((B * NB, 1, NR), jnp.float32),
            jax.ShapeDtypeStruct((B * NB, 1, NR), jnp.int32),
        ],
    )(logits)
    return m.reshape(B, N), c1.reshape(B, N)


def kernel(pred_logits, pred_boxes, orig_sizes):
    scores_all, labels = _dense_reduce(pred_logits)

    top_scores, top_idx = lax.top_k(scores_all, K)
    top_labels = jnp.take_along_axis(labels, top_idx, axis=1)
    top_boxes = jnp.take_along_axis(pred_boxes, top_idx[..., None], axis=1)
    cx, cy, w, h = (top_boxes[..., i] for i in range(4))
    box_xyxy = jnp.stack(
        [cx - 0.5 * w, cy - 0.5 * h, cx + 0.5 * w, cy + 0.5 * h], axis=-1)
    hw = orig_sizes.astype(jnp.float32)
    scale = jnp.stack([hw[:, 1], hw[:, 0], hw[:, 1], hw[:, 0]],
                      axis=-1)[:, None, :]
    return top_scores, top_labels, box_xyxy * scale
